# baseline (speedup 1.0000x reference)
import functools

import jax
import jax.numpy as jnp
from jax.experimental import pallas as pl
from jax.experimental.pallas import tpu as pltpu

D_PROJ = 1024
CUT0, CUT1, CUT2 = 20000, 60000, 100000
TBS = 128


def _dotT(a, b):
    return jax.lax.dot_general(a, b, (((1,), (1,)), ((), ())),
                               preferred_element_type=jnp.float32)


def _dot(a, b):
    return jax.lax.dot_general(a, b, (((1,), (0,)), ((), ())),
                               preferred_element_type=jnp.float32)


def _route_body(t_row_ref, h_ref, proj1_ref, proj2_ref,
                pos_ref, ts_ref, cnt_ref, ph1s_ref, ph2s_ref):
    T = h_ref.shape[0]
    t = t_row_ref[...]
    cid = (t >= CUT0).astype(jnp.int32) + (t >= CUT1).astype(jnp.int32)
    rid = jax.lax.broadcasted_iota(jnp.int32, (8, T), 0)
    mask8 = (rid == cid).astype(jnp.bfloat16)
    ri = jax.lax.broadcasted_iota(jnp.int32, (T, T), 0)
    ci = jax.lax.broadcasted_iota(jnp.int32, (T, T), 1)
    tri = (ri <= ci).astype(jnp.bfloat16)
    cums = _dot(mask8, tri)
    n0 = cums[0:1, T - 1:T]
    n1 = cums[1:2, T - 1:T]
    cnt_ref[...] = jnp.concatenate(
        [n0, n1, cums[2:3, T - 1:T],
         jnp.zeros((1, 5), jnp.float32)], axis=1).astype(jnp.int32)
    m0 = mask8[0:1].astype(jnp.float32)
    m1 = mask8[1:2].astype(jnp.float32)
    m2 = mask8[2:3].astype(jnp.float32)
    own = m0 * cums[0:1] + m1 * cums[1:2] + m2 * cums[2:3]
    off = m1 * n0 + m2 * (n0 + n1)
    pos = (own + off - 1.0).astype(jnp.int32)
    pos_ref[...] = pos
    rj = jax.lax.broadcasted_iota(jnp.int32, (T, T), 0)
    a_t = (rj == pos).astype(jnp.bfloat16)
    byts = jnp.concatenate(
        [(t // 65536).astype(jnp.bfloat16),
         ((t // 256) % 256).astype(jnp.bfloat16),
         (t % 256).astype(jnp.bfloat16),
         jnp.zeros((5, T), jnp.bfloat16)], axis=0)
    tsb = jax.lax.dot_general(byts, a_t, (((1,), (1,)), ((), ())),
                              preferred_element_type=jnp.float32)
    ts = (tsb[0:1].astype(jnp.int32) * 65536
          + tsb[1:2].astype(jnp.int32) * 256 + tsb[2:3].astype(jnp.int32))
    ts_ref[...] = ts
    ph1 = _dot(h_ref[...], proj1_ref[...]).astype(jnp.bfloat16)
    ph2 = _dot(h_ref[...], proj2_ref[...]).astype(jnp.bfloat16)
    ph1s_ref[...] = _dot(a_t, ph1).astype(jnp.bfloat16)
    ph2s_ref[...] = _dot(a_t, ph2).astype(jnp.bfloat16)


def _route_call(t_row, h, proj1, proj2):
    T, D = h.shape
    k1, k2 = proj1.shape[1], proj2.shape[1]
    full = lambda *s: pl.BlockSpec(s, lambda: (0,) * len(s))
    return pl.pallas_call(
        _route_body,
        in_specs=[full(1, T), full(T, D), full(D, k1), full(D, k2)],
        out_specs=[full(1, T), full(1, T), full(1, 8),
                   full(T, k1), full(T, k2)],
        out_shape=[jax.ShapeDtypeStruct((1, T), jnp.int32),
                   jax.ShapeDtypeStruct((1, T), jnp.int32),
                   jax.ShapeDtypeStruct((1, 8), jnp.int32),
                   jax.ShapeDtypeStruct((T, k1), jnp.bfloat16),
                   jax.ShapeDtypeStruct((T, k2), jnp.bfloat16)],
    )(t_row, h, proj1, proj2)


def _head_body(h_ref, proj_ref, w_ref, b_ref, t_ref,
               s_ref, tg_ref, ca_ref, cb_ref, ph_scr,
               *, v_tile, c_pos):
    v = pl.program_id(0)

    @pl.when(v == 0)
    def _init():
        ph_scr[...] = _dot(h_ref[...], proj_ref[...]).astype(jnp.bfloat16)
        s_ref[...] = jnp.zeros_like(s_ref)
        tg_ref[...] = jnp.zeros_like(tg_ref)
        ca_ref[...] = jnp.zeros_like(ca_ref)
        cb_ref[...] = jnp.zeros_like(cb_ref)

    logits = _dotT(ph_scr[...], w_ref[...]) + b_ref[...]
    s_ref[...] += jnp.sum(jnp.exp(logits), axis=1, keepdims=True)
    local_t = t_ref[...] - v * v_tile
    cols = jax.lax.broadcasted_iota(jnp.int32, logits.shape, 1)
    tg_ref[...] += jnp.sum(jnp.where(cols == local_t, logits, 0.0),
                           axis=1, keepdims=True)
    c_tile, c_loc = c_pos

    @pl.when(v == c_tile)
    def _cl():
        ca_ref[...] = logits[:, c_loc:c_loc + 1]
        cb_ref[...] = logits[:, c_loc + 1:c_loc + 2]


def _head_call(h, proj, w, b, t2d, v_tile, c_pos):
    T, D = h.shape
    K = proj.shape[1]
    n_tiles = w.shape[0] // v_tile
    full = lambda *s: pl.BlockSpec(s, lambda v: (0,) * len(s))
    out1 = jax.ShapeDtypeStruct((T, 1), jnp.float32)
    return pl.pallas_call(
        functools.partial(_head_body, v_tile=v_tile, c_pos=c_pos),
        grid=(n_tiles,),
        in_specs=[
            full(T, D), full(D, K),
            pl.BlockSpec((v_tile, K), lambda v: (v, 0)),
            pl.BlockSpec((1, v_tile), lambda v: (0, v)),
            full(T, 1),
        ],
        out_specs=[full(T, 1)] * 4,
        out_shape=[out1] * 4,
        scratch_shapes=[pltpu.VMEM((T, K), jnp.bfloat16)],
        compiler_params=pltpu.CompilerParams(
            dimension_semantics=("arbitrary",)),
    )(h, proj, w, b, t2d)


def _tail_body(cnt_ref, ph_ref, w_ref, b_ref, ts_ref, s_ref, tg_ref,
               *, v_tile, t_off, cluster):
    v = pl.program_id(0)

    @pl.when(v == 0)
    def _init():
        s_ref[...] = jnp.zeros_like(s_ref)
        tg_ref[...] = jnp.zeros_like(tg_ref)

    n0 = cnt_ref[0, 0]
    n1 = cnt_ref[0, 1]
    if cluster == 1:
        start, end = n0, n0 + n1
    else:
        start, end = n0 + n1, n0 + n1 + cnt_ref[0, 2]
    blk0 = start // TBS
    blk1 = (end + TBS - 1) // TBS
    cols = jax.lax.broadcasted_iota(jnp.int32, (TBS, v_tile), 1)

    def _blk(tb, carry):
        rows = pl.ds(tb * TBS, TBS)
        logits = _dotT(ph_ref[rows, :], w_ref[...]) + b_ref[...]
        s_ref[rows, :] += jnp.sum(jnp.exp(logits), axis=1, keepdims=True)
        local_t = ts_ref[rows, :] - (t_off + v * v_tile)
        tg_ref[rows, :] += jnp.sum(jnp.where(cols == local_t, logits, 0.0),
                                   axis=1, keepdims=True)
        return carry

    jax.lax.fori_loop(blk0, blk1, _blk, 0)


def _tail_call(cnt, ph_s, w, b, ts_col, v_tile, t_off, cluster):
    T, K = ph_s.shape
    n_tiles = w.shape[0] // v_tile
    full = lambda *s: pl.BlockSpec(s, lambda v: (0,) * len(s))
    out1 = jax.ShapeDtypeStruct((T, 1), jnp.float32)
    return pl.pallas_call(
        functools.partial(_tail_body, v_tile=v_tile, t_off=t_off,
                          cluster=cluster),
        grid=(n_tiles,),
        in_specs=[
            pl.BlockSpec(memory_space=pltpu.SMEM),
            full(T, K),
            pl.BlockSpec((v_tile, K), lambda v: (v, 0)),
            pl.BlockSpec((1, v_tile), lambda v: (0, v)),
            full(T, 1),
        ],
        out_specs=[full(T, 1)] * 2,
        out_shape=[out1] * 2,
        compiler_params=pltpu.CompilerParams(
            dimension_semantics=("arbitrary",)),
    )(cnt, ph_s, w, b, ts_col)


def _combine_body(t_ref, pos_ref, s0_ref, tg0_ref, ca_ref, cb_ref,
                  s1_ref, tg1_ref, s2_ref, tg2_ref, nll_ref):
    T = t_ref.shape[0]
    t = t_ref[...]
    pos = pos_ref[...]
    cols = jax.lax.broadcasted_iota(jnp.int32, (T, T), 1)
    bmat = (cols == pos).astype(jnp.bfloat16)
    packed = jnp.concatenate(
        [s1_ref[...], tg1_ref[...], s2_ref[...], tg2_ref[...],
         jnp.zeros((T, 124), jnp.float32)], axis=1).astype(jnp.bfloat16)
    back = _dot(bmat, packed)
    s1, tg1 = back[:, 0:1], back[:, 1:2]
    s2, tg2 = back[:, 2:3], back[:, 3:4]
    lse0 = jnp.log(s0_ref[...])
    nll0 = lse0 - tg0_ref[...]
    nll1 = lse0 - cb_ref[...] + jnp.log(s1) - tg1
    nll2 = lse0 - ca_ref[...] + jnp.log(s2) - tg2
    nll_ref[...] = jnp.where(t < CUT0, nll0, jnp.where(t < CUT1, nll1, nll2))


def _prep(w, b, v_pad):
    v = w.shape[0]
    wp = jnp.pad(w, ((0, v_pad - v), (0, 0))).astype(jnp.bfloat16)
    bp = jnp.pad(b, (0, v_pad - v), constant_values=-1e9).reshape(1, -1)
    return wp, bp.astype(jnp.float32)


def kernel(hidden, target, w0, b0, cluster_w, cluster_b, proj0,
           w1, b1, proj1, w2, b2, proj2):
    tgt_shape = target.shape
    h = hidden.reshape(-1, D_PROJ).astype(jnp.bfloat16)
    t_row = target.reshape(1, -1).astype(jnp.int32)
    t2d = target.reshape(-1, 1).astype(jnp.int32)
    T = h.shape[0]

    w0p, b0p = _prep(jnp.concatenate([w0, cluster_w], axis=0),
                     jnp.concatenate([b0, cluster_b], axis=0), 20480)
    w1p, b1p = _prep(w1, b1, 40960)
    w2p, b2p = _prep(w2, b2, 40960)

    pos_row, ts_row, cnt, ph1s, ph2s = _route_call(
        t_row, h, proj1.astype(jnp.bfloat16), proj2.astype(jnp.bfloat16))
    ts_col = ts_row.reshape(T, 1)
    pos_col = pos_row.reshape(T, 1)

    vt_h, vt_t = 1024, 4096
    c_pos = (CUT0 // vt_h, CUT0 % vt_h)
    s0, tg0, ca, cb = _head_call(h, proj0.astype(jnp.bfloat16), w0p, b0p,
                                 t2d, vt_h, c_pos)
    s1, tg1 = _tail_call(cnt, ph1s, w1p, b1p, ts_col, vt_t, CUT0, 1)
    s2, tg2 = _tail_call(cnt, ph2s, w2p, b2p, ts_col, vt_t, CUT1, 2)

    full = pl.BlockSpec((T, 1), lambda: (0, 0))
    nll = pl.pallas_call(
        _combine_body,
        in_specs=[full] * 10,
        out_specs=full,
        out_shape=jax.ShapeDtypeStruct((T, 1), jnp.float32),
    )(t2d, pos_col, s0, tg0, ca, cb, s1, tg1, s2, tg2)
    return nll.reshape(tgt_shape)


# speedup vs baseline: 1.0478x; 1.0478x over previous
import functools

import jax
import jax.numpy as jnp
from jax.experimental import pallas as pl
from jax.experimental.pallas import tpu as pltpu

D_PROJ = 1024
CUT0, CUT1, CUT2 = 20000, 60000, 100000
TBS = 256


def _dotT(a, b):
    return jax.lax.dot_general(a, b, (((1,), (1,)), ((), ())),
                               preferred_element_type=jnp.float32)


def _dot(a, b):
    return jax.lax.dot_general(a, b, (((1,), (0,)), ((), ())),
                               preferred_element_type=jnp.float32)


def _route_body(t_row_ref, h_ref, proj1_ref, proj2_ref,
                pos_ref, ts_ref, cnt_ref, ph1s_ref, ph2s_ref):
    T = h_ref.shape[0]
    t = t_row_ref[...]
    cid = (t >= CUT0).astype(jnp.int32) + (t >= CUT1).astype(jnp.int32)
    rid = jax.lax.broadcasted_iota(jnp.int32, (8, T), 0)
    mask8 = (rid == cid).astype(jnp.bfloat16)
    ri = jax.lax.broadcasted_iota(jnp.int32, (T, T), 0)
    ci = jax.lax.broadcasted_iota(jnp.int32, (T, T), 1)
    tri = (ri <= ci).astype(jnp.bfloat16)
    cums = _dot(mask8, tri)
    n0 = cums[0:1, T - 1:T]
    n1 = cums[1:2, T - 1:T]
    cnt_ref[...] = jnp.concatenate(
        [n0, n1, cums[2:3, T - 1:T],
         jnp.zeros((1, 5), jnp.float32)], axis=1).astype(jnp.int32)
    m0 = mask8[0:1].astype(jnp.float32)
    m1 = mask8[1:2].astype(jnp.float32)
    m2 = mask8[2:3].astype(jnp.float32)
    own = m0 * cums[0:1] + m1 * cums[1:2] + m2 * cums[2:3]
    off = m1 * n0 + m2 * (n0 + n1)
    pos = (own + off - 1.0).astype(jnp.int32)
    pos_ref[...] = pos
    rj = jax.lax.broadcasted_iota(jnp.int32, (T, T), 0)
    a_t = (rj == pos).astype(jnp.bfloat16)
    byts = jnp.concatenate(
        [(t // 65536).astype(jnp.bfloat16),
         ((t // 256) % 256).astype(jnp.bfloat16),
         (t % 256).astype(jnp.bfloat16),
         jnp.zeros((5, T), jnp.bfloat16)], axis=0)
    tsb = jax.lax.dot_general(byts, a_t, (((1,), (1,)), ((), ())),
                              preferred_element_type=jnp.float32)
    ts = (tsb[0:1].astype(jnp.int32) * 65536
          + tsb[1:2].astype(jnp.int32) * 256 + tsb[2:3].astype(jnp.int32))
    ts_ref[...] = ts
    ph1 = _dot(h_ref[...], proj1_ref[...]).astype(jnp.bfloat16)
    ph2 = _dot(h_ref[...], proj2_ref[...]).astype(jnp.bfloat16)
    ph1s_ref[...] = _dot(a_t, ph1).astype(jnp.bfloat16)
    ph2s_ref[...] = _dot(a_t, ph2).astype(jnp.bfloat16)


def _route_call(t_row, h, proj1, proj2):
    T, D = h.shape
    k1, k2 = proj1.shape[1], proj2.shape[1]
    full = lambda *s: pl.BlockSpec(s, lambda: (0,) * len(s))
    return pl.pallas_call(
        _route_body,
        in_specs=[full(1, T), full(T, D), full(D, k1), full(D, k2)],
        out_specs=[full(1, T), full(1, T), full(1, 8),
                   full(T, k1), full(T, k2)],
        out_shape=[jax.ShapeDtypeStruct((1, T), jnp.int32),
                   jax.ShapeDtypeStruct((1, T), jnp.int32),
                   jax.ShapeDtypeStruct((1, 8), jnp.int32),
                   jax.ShapeDtypeStruct((T, k1), jnp.bfloat16),
                   jax.ShapeDtypeStruct((T, k2), jnp.bfloat16)],
    )(t_row, h, proj1, proj2)


def _head_body(h_ref, proj_ref, w_ref, b_ref, t_ref,
               s_ref, tg_ref, ca_ref, cb_ref, ph_scr,
               *, v_tile, c_pos):
    v = pl.program_id(0)

    @pl.when(v == 0)
    def _init():
        ph_scr[...] = _dot(h_ref[...], proj_ref[...]).astype(jnp.bfloat16)
        s_ref[...] = jnp.zeros_like(s_ref)
        tg_ref[...] = jnp.zeros_like(tg_ref)
        ca_ref[...] = jnp.zeros_like(ca_ref)
        cb_ref[...] = jnp.zeros_like(cb_ref)

    logits = _dotT(ph_scr[...], w_ref[...]) + b_ref[...]
    s_ref[...] += jnp.sum(jnp.exp(logits), axis=1, keepdims=True)
    local_t = t_ref[...] - v * v_tile
    cols = jax.lax.broadcasted_iota(jnp.int32, logits.shape, 1)
    tg_ref[...] += jnp.sum(jnp.where(cols == local_t, logits, 0.0),
                           axis=1, keepdims=True)
    c_tile, c_loc = c_pos

    @pl.when(v == c_tile)
    def _cl():
        ca_ref[...] = logits[:, c_loc:c_loc + 1]
        cb_ref[...] = logits[:, c_loc + 1:c_loc + 2]


def _head_call(h, proj, w, b, t2d, v_tile, c_pos):
    T, D = h.shape
    K = proj.shape[1]
    n_tiles = w.shape[0] // v_tile
    full = lambda *s: pl.BlockSpec(s, lambda v: (0,) * len(s))
    out1 = jax.ShapeDtypeStruct((T, 1), jnp.float32)
    return pl.pallas_call(
        functools.partial(_head_body, v_tile=v_tile, c_pos=c_pos),
        grid=(n_tiles,),
        in_specs=[
            full(T, D), full(D, K),
            pl.BlockSpec((v_tile, K), lambda v: (v, 0)),
            pl.BlockSpec((1, v_tile), lambda v: (0, v)),
            full(T, 1),
        ],
        out_specs=[full(T, 1)] * 4,
        out_shape=[out1] * 4,
        scratch_shapes=[pltpu.VMEM((T, K), jnp.bfloat16)],
        compiler_params=pltpu.CompilerParams(
            dimension_semantics=("arbitrary",)),
    )(h, proj, w, b, t2d)


def _tail_body(cnt_ref, ph_ref, w_ref, b_ref, ts_ref, s_ref, tg_ref,
               *, v_tile, t_off, cluster):
    v = pl.program_id(0)

    @pl.when(v == 0)
    def _init():
        s_ref[...] = jnp.zeros_like(s_ref)
        tg_ref[...] = jnp.zeros_like(tg_ref)

    n0 = cnt_ref[0, 0]
    n1 = cnt_ref[0, 1]
    if cluster == 1:
        start, end = n0, n0 + n1
    else:
        start, end = n0 + n1, n0 + n1 + cnt_ref[0, 2]
    blk0 = start // TBS
    blk1 = (end + TBS - 1) // TBS
    cols = jax.lax.broadcasted_iota(jnp.int32, (TBS, v_tile), 1)

    def _blk(tb, carry):
        rows = pl.ds(tb * TBS, TBS)
        logits = _dotT(ph_ref[rows, :], w_ref[...]) + b_ref[...]
        s_ref[rows, :] += jnp.sum(jnp.exp(logits), axis=1, keepdims=True)
        local_t = ts_ref[rows, :] - (t_off + v * v_tile)
        tg_ref[rows, :] += jnp.sum(jnp.where(cols == local_t, logits, 0.0),
                                   axis=1, keepdims=True)
        return carry

    jax.lax.fori_loop(blk0, blk1, _blk, 0)


def _tail_call(cnt, ph_s, w, b, ts_col, v_tile, t_off, cluster):
    T, K = ph_s.shape
    n_tiles = w.shape[0] // v_tile
    full = lambda *s: pl.BlockSpec(s, lambda v: (0,) * len(s))
    out1 = jax.ShapeDtypeStruct((T, 1), jnp.float32)
    return pl.pallas_call(
        functools.partial(_tail_body, v_tile=v_tile, t_off=t_off,
                          cluster=cluster),
        grid=(n_tiles,),
        in_specs=[
            pl.BlockSpec(memory_space=pltpu.SMEM),
            full(T, K),
            pl.BlockSpec((v_tile, K), lambda v: (v, 0)),
            pl.BlockSpec((1, v_tile), lambda v: (0, v)),
            full(T, 1),
        ],
        out_specs=[full(T, 1)] * 2,
        out_shape=[out1] * 2,
        compiler_params=pltpu.CompilerParams(
            dimension_semantics=("arbitrary",)),
    )(cnt, ph_s, w, b, ts_col)


def _combine_body(t_ref, pos_ref, s0_ref, tg0_ref, ca_ref, cb_ref,
                  s1_ref, tg1_ref, s2_ref, tg2_ref, nll_ref):
    T = t_ref.shape[0]
    t = t_ref[...]
    pos = pos_ref[...]
    cols = jax.lax.broadcasted_iota(jnp.int32, (T, T), 1)
    bmat = (cols == pos).astype(jnp.bfloat16)
    packed = jnp.concatenate(
        [s1_ref[...], tg1_ref[...], s2_ref[...], tg2_ref[...],
         jnp.zeros((T, 124), jnp.float32)], axis=1).astype(jnp.bfloat16)
    back = _dot(bmat, packed)
    s1, tg1 = back[:, 0:1], back[:, 1:2]
    s2, tg2 = back[:, 2:3], back[:, 3:4]
    lse0 = jnp.log(s0_ref[...])
    nll0 = lse0 - tg0_ref[...]
    nll1 = lse0 - cb_ref[...] + jnp.log(s1) - tg1
    nll2 = lse0 - ca_ref[...] + jnp.log(s2) - tg2
    nll_ref[...] = jnp.where(t < CUT0, nll0, jnp.where(t < CUT1, nll1, nll2))


def _prep(w, b, v_pad):
    v = w.shape[0]
    wp = jnp.pad(w, ((0, v_pad - v), (0, 0))).astype(jnp.bfloat16)
    bp = jnp.pad(b, (0, v_pad - v), constant_values=-1e9).reshape(1, -1)
    return wp, bp.astype(jnp.float32)


def kernel(hidden, target, w0, b0, cluster_w, cluster_b, proj0,
           w1, b1, proj1, w2, b2, proj2):
    tgt_shape = target.shape
    h = hidden.reshape(-1, D_PROJ).astype(jnp.bfloat16)
    t_row = target.reshape(1, -1).astype(jnp.int32)
    t2d = target.reshape(-1, 1).astype(jnp.int32)
    T = h.shape[0]

    w0p, b0p = _prep(jnp.concatenate([w0, cluster_w], axis=0),
                     jnp.concatenate([b0, cluster_b], axis=0), 20480)
    w1p, b1p = _prep(w1, b1, 40960)
    w2p, b2p = _prep(w2, b2, 40960)

    pos_row, ts_row, cnt, ph1s, ph2s = _route_call(
        t_row, h, proj1.astype(jnp.bfloat16), proj2.astype(jnp.bfloat16))
    ts_col = ts_row.reshape(T, 1)
    pos_col = pos_row.reshape(T, 1)

    vt_h, vt_t = 1024, 8192
    c_pos = (CUT0 // vt_h, CUT0 % vt_h)
    s0, tg0, ca, cb = _head_call(h, proj0.astype(jnp.bfloat16), w0p, b0p,
                                 t2d, vt_h, c_pos)
    s1, tg1 = _tail_call(cnt, ph1s, w1p, b1p, ts_col, vt_t, CUT0, 1)
    s2, tg2 = _tail_call(cnt, ph2s, w2p, b2p, ts_col, vt_t, CUT1, 2)

    full = pl.BlockSpec((T, 1), lambda: (0, 0))
    nll = pl.pallas_call(
        _combine_body,
        in_specs=[full] * 10,
        out_specs=full,
        out_shape=jax.ShapeDtypeStruct((T, 1), jnp.float32),
    )(t2d, pos_col, s0, tg0, ca, cb, s1, tg1, s2, tg2)
    return nll.reshape(tgt_shape)
